# baseline (speedup 1.0000x reference)
import functools

import jax
import jax.numpy as jnp
from jax.experimental import pallas as pl
from jax.experimental.pallas import tpu as pltpu

_EPS = 1e-5


def _fused_kernel(x_ref, wemb_ref, g_in_ref, b_in_ref, wgcn_ref, adj_ref,
                  g_loc_ref, b_loc_ref, out_ref, big_ref, sup_ref,
                  acc_ref, *, nb1, b1, nb2, b2, nb3, b3):
    xemb_ref = big_ref
    xgcn_ref = big_ref
    i = pl.program_id(0)
    n_rows = float(nb1 * b1)
    p1, p2, p3 = nb1, 2 * nb1, 2 * nb1 + nb2

    @pl.when(i == 0)
    def _():
        acc_ref[...] = jnp.zeros_like(acc_ref)

    @pl.when(i < p1)
    def _():
        xe = jnp.dot(x_ref[...], wemb_ref[...],
                     preferred_element_type=jnp.float32)
        j = jnp.minimum(i, nb1 - 1)
        xemb_ref[pl.ds(j * b1, b1), :] = xe
        acc_ref[0:1, :] += jnp.sum(xe, axis=0, keepdims=True)
        acc_ref[1:2, :] += jnp.sum(xe * xe, axis=0, keepdims=True)

    @pl.when(jnp.logical_and(i >= p1, i < p2))
    def _():
        mu = acc_ref[0:1, :] / n_rows
        var = acc_ref[1:2, :] / n_rows - mu * mu
        a = g_in_ref[...] * jax.lax.rsqrt(var + _EPS)
        b = b_in_ref[...] - mu * a
        j = jnp.clip(i - p1, 0, nb1 - 1)
        h = jnp.maximum(xemb_ref[pl.ds(j * b1, b1), :] * a + b, 0.0)
        sup_ref[pl.ds(j * b1, b1), :] = jnp.dot(
            h, wgcn_ref[...], preferred_element_type=jnp.float32
        ).astype(jnp.bfloat16)

    @pl.when(jnp.logical_and(i >= p2, i < p3))
    def _():
        @pl.when(i == p2)
        def _():
            acc_ref[...] = jnp.zeros_like(acc_ref)

        a = adj_ref[...].astype(jnp.bfloat16)
        xg = jnp.dot(a, sup_ref[...], preferred_element_type=jnp.float32)
        j = jnp.clip(i - p2, 0, nb2 - 1)
        xgcn_ref[pl.ds(j * b2, b2), :] = xg
        acc_ref[0:1, :] += jnp.sum(xg, axis=0, keepdims=True)
        acc_ref[1:2, :] += jnp.sum(xg * xg, axis=0, keepdims=True)

    @pl.when(i >= p3)
    def _():
        mu = acc_ref[0:1, :] / n_rows
        var = acc_ref[1:2, :] / n_rows - mu * mu
        a2 = g_loc_ref[...] * jax.lax.rsqrt(var + _EPS)
        b2 = b_loc_ref[...] - mu * a2
        j = jnp.clip(i - p3, 0, nb3 - 1)
        out_ref[...] = xgcn_ref[pl.ds(j * b3, b3), :] * a2 + b2


def kernel(x, adj, W_emb, gcn_weight, gamma_in, beta_in, gamma_local,
           beta_local):
    N, F = x.shape
    D = W_emb.shape[1]
    g_in = gamma_in.reshape(1, D)
    b_in = beta_in.reshape(1, D)
    g_loc = gamma_local.reshape(1, D)
    b_loc = beta_local.reshape(1, D)

    B1 = 2000
    NB1 = N // B1
    B2 = 400
    NB2 = N // B2
    B3 = 1000
    NB3 = N // B3
    p2, p3 = 2 * NB1, 2 * NB1 + NB2
    grid = 2 * NB1 + NB2 + NB3

    out = pl.pallas_call(
        functools.partial(_fused_kernel, nb1=NB1, b1=B1, nb2=NB2, b2=B2,
                          nb3=NB3, b3=B3),
        grid=(grid,),
        in_specs=[
            pl.BlockSpec((B1, F), lambda i: (jnp.minimum(i, NB1 - 1), 0)),
            pl.BlockSpec((F, D), lambda i: (0, 0)),
            pl.BlockSpec((1, D), lambda i: (0, 0)),
            pl.BlockSpec((1, D), lambda i: (0, 0)),
            pl.BlockSpec((D, D), lambda i: (0, 0)),
            pl.BlockSpec((B2, N), lambda i: (jnp.clip(i - p2, 0, NB2 - 1), 0)),
            pl.BlockSpec((1, D), lambda i: (0, 0)),
            pl.BlockSpec((1, D), lambda i: (0, 0)),
        ],
        out_specs=pl.BlockSpec((B3, D),
                               lambda i: (jnp.clip(i - p3, 0, NB3 - 1), 0)),
        out_shape=jax.ShapeDtypeStruct((N, D), jnp.float32),
        scratch_shapes=[
            pltpu.VMEM((N, D), jnp.float32),
            pltpu.VMEM((N, D), jnp.bfloat16),
            pltpu.VMEM((2, D), jnp.float32),
        ],
        compiler_params=pltpu.CompilerParams(
            dimension_semantics=("arbitrary",),
            vmem_limit_bytes=64 * 1024 * 1024,
        ),
    )(x, W_emb, g_in, b_in, gcn_weight, adj, g_loc, b_loc)

    return out


# speedup vs baseline: 1.0088x; 1.0088x over previous
import functools

import jax
import jax.numpy as jnp
from jax.experimental import pallas as pl
from jax.experimental.pallas import tpu as pltpu

_EPS = 1e-5


def _fused_kernel(x_ref, wemb_ref, g_in_ref, b_in_ref, wgcn_ref, adj_ref,
                  g_loc_ref, b_loc_ref, out_ref, xemb_ref, sup_ref, xgcn_ref,
                  acc_ref, *, nb1, b1, nb2, b2, nb3, b3):
    i = pl.program_id(0)
    n_rows = float(nb1 * b1)
    p1, p2, p3 = nb1, 2 * nb1, 2 * nb1 + nb2

    @pl.when(i == 0)
    def _():
        acc_ref[...] = jnp.zeros_like(acc_ref)

    @pl.when(i < p1)
    def _():
        xe = jnp.dot(x_ref[...], wemb_ref[...],
                     preferred_element_type=jnp.float32)
        j = jnp.minimum(i, nb1 - 1)
        xemb_ref[pl.ds(j * b1, b1), :] = xe
        acc_ref[0:1, :] += jnp.sum(xe, axis=0, keepdims=True)
        acc_ref[1:2, :] += jnp.sum(xe * xe, axis=0, keepdims=True)

    @pl.when(jnp.logical_and(i >= p1, i < p2))
    def _():
        mu = acc_ref[0:1, :] / n_rows
        var = acc_ref[1:2, :] / n_rows - mu * mu
        a = g_in_ref[...] * jax.lax.rsqrt(var + _EPS)
        b = b_in_ref[...] - mu * a
        j = jnp.clip(i - p1, 0, nb1 - 1)
        h = jnp.maximum(xemb_ref[pl.ds(j * b1, b1), :] * a + b, 0.0)
        sup_ref[pl.ds(j * b1, b1), :] = jnp.dot(
            h, wgcn_ref[...], preferred_element_type=jnp.float32
        ).astype(jnp.bfloat16)

    @pl.when(jnp.logical_and(i >= p2, i < p3))
    def _():
        @pl.when(i == p2)
        def _():
            acc_ref[...] = jnp.zeros_like(acc_ref)

        a = adj_ref[...].astype(jnp.bfloat16)
        xg = jnp.dot(a, sup_ref[...], preferred_element_type=jnp.float32)
        j = jnp.clip(i - p2, 0, nb2 - 1)
        xgcn_ref[pl.ds(j * b2, b2), :] = xg
        acc_ref[0:1, :] += jnp.sum(xg, axis=0, keepdims=True)
        acc_ref[1:2, :] += jnp.sum(xg * xg, axis=0, keepdims=True)

    @pl.when(i >= p3)
    def _():
        mu = acc_ref[0:1, :] / n_rows
        var = acc_ref[1:2, :] / n_rows - mu * mu
        a2 = g_loc_ref[...] * jax.lax.rsqrt(var + _EPS)
        b2 = b_loc_ref[...] - mu * a2
        j = jnp.clip(i - p3, 0, nb3 - 1)
        out_ref[...] = xgcn_ref[pl.ds(j * b3, b3), :] * a2 + b2


def kernel(x, adj, W_emb, gcn_weight, gamma_in, beta_in, gamma_local,
           beta_local):
    N, F = x.shape
    D = W_emb.shape[1]
    g_in = gamma_in.reshape(1, D)
    b_in = beta_in.reshape(1, D)
    g_loc = gamma_local.reshape(1, D)
    b_loc = beta_local.reshape(1, D)

    B1 = 2000
    NB1 = N // B1
    B2 = 400
    NB2 = N // B2
    B3 = 1000
    NB3 = N // B3
    p2, p3 = 2 * NB1, 2 * NB1 + NB2
    grid = 2 * NB1 + NB2 + NB3

    out = pl.pallas_call(
        functools.partial(_fused_kernel, nb1=NB1, b1=B1, nb2=NB2, b2=B2,
                          nb3=NB3, b3=B3),
        grid=(grid,),
        in_specs=[
            pl.BlockSpec((B1, F), lambda i: (jnp.minimum(i, NB1 - 1), 0)),
            pl.BlockSpec((F, D), lambda i: (0, 0)),
            pl.BlockSpec((1, D), lambda i: (0, 0)),
            pl.BlockSpec((1, D), lambda i: (0, 0)),
            pl.BlockSpec((D, D), lambda i: (0, 0)),
            pl.BlockSpec((B2, N), lambda i: (jnp.clip(i - p2, 0, NB2 - 1), 0)),
            pl.BlockSpec((1, D), lambda i: (0, 0)),
            pl.BlockSpec((1, D), lambda i: (0, 0)),
        ],
        out_specs=pl.BlockSpec((B3, D),
                               lambda i: (jnp.clip(i - p3, 0, NB3 - 1), 0)),
        out_shape=jax.ShapeDtypeStruct((N, D), jnp.float32),
        scratch_shapes=[
            pltpu.VMEM((N, D), jnp.float32),
            pltpu.VMEM((N, D), jnp.bfloat16),
            pltpu.VMEM((N, D), jnp.float32),
            pltpu.VMEM((2, D), jnp.float32),
        ],
        compiler_params=pltpu.CompilerParams(
            dimension_semantics=("arbitrary",),
            vmem_limit_bytes=64 * 1024 * 1024,
        ),
    )(x, W_emb, g_in, b_in, gcn_weight, adj, g_loc, b_loc)

    return out
